# baseline (speedup 1.0000x reference)
import functools

import jax
import jax.numpy as jnp
from jax import lax
from jax.experimental import pallas as pl
from jax.experimental.pallas import tpu as pltpu
from jax.experimental.pallas import tpu_sc as plsc

N = 10000
E = 160000
NP = 10240
NC = 2
NS = 16
NW = NC * NS
CH = 128
NCHUNK = 40
EPW = CH * NCHUNK
EP = EPW * NW
STRIPE = NP // NS

_MESH = plsc.VectorSubcoreMesh(core_axis_name="c", subcore_axis_name="s",
                               num_cores=NC, num_subcores=NS)


def _deg_body(dst_hbm, w_hbm, out_hbm, dst_v, w_v, zv, acc_sh, sem):
    cid = lax.axis_index("c")
    sid = lax.axis_index("s")
    wid = cid * NS + sid
    row0 = sid * STRIPE

    def zstore(r, carry):
        zv[pl.ds(r * 16, 16)] = jnp.zeros((16,), jnp.float32)
        return carry

    lax.fori_loop(0, STRIPE // 16, zstore, 0)
    pltpu.sync_copy(zv, acc_sh.at[pl.ds(row0, STRIPE)])
    pltpu.sync_copy(dst_hbm.at[wid], dst_v)
    pltpu.sync_copy(w_hbm.at[wid], w_v)
    plsc.subcore_barrier()
    def chunk(j, carry):
        pltpu.async_copy(w_v.at[j], acc_sh.at[dst_v.at[j]], sem, add=True)
        return carry

    lax.fori_loop(0, NCHUNK, chunk, 0)

    def drain(j, carry):
        pltpu.make_async_copy(w_v.at[0], acc_sh.at[dst_v.at[0]], sem).wait()
        return carry

    lax.fori_loop(0, NCHUNK, drain, 0)
    plsc.subcore_barrier()
    pltpu.sync_copy(acc_sh.at[pl.ds(row0, STRIPE)],
                    out_hbm.at[cid, pl.ds(row0, STRIPE)])


def _sc_degree(dst3, w3):
    return pl.kernel(
        _deg_body,
        out_type=jax.ShapeDtypeStruct((NC, NP), jnp.float32),
        mesh=_MESH,
        scratch_types=[
            pltpu.VMEM((NCHUNK, CH), jnp.int32),
            pltpu.VMEM((NCHUNK, CH), jnp.float32),
            pltpu.VMEM((STRIPE,), jnp.float32),
            pltpu.VMEM_SHARED((NP,), jnp.float32),
            pltpu.SemaphoreType.DMA,
        ],
    )(dst3, w3)


def _mp_body(xsp_hbm, srcf_hbm, dst_hbm, w_hbm, out_hbm,
             srcf_v, dst_v, w_v, bigs, gsems, ssems, xs_sh, acc_sh, D, PC):
    PE = PC * CH
    NPH = NCHUNK // PC
    cid = lax.axis_index("c")
    sid = lax.axis_index("s")
    wid = cid * NS + sid
    row0 = sid * STRIPE
    pltpu.sync_copy(srcf_hbm.at[wid], srcf_v)
    pltpu.sync_copy(dst_hbm.at[wid], dst_v)
    pltpu.sync_copy(w_hbm.at[wid], w_v)
    NSTR = N // NS
    pltpu.sync_copy(xsp_hbm.at[pl.ds(sid * NSTR, NSTR)],
                    xs_sh.at[pl.ds(sid * NSTR, NSTR)])
    ZR = 128

    def zstore(r, carry):
        for f in range(D // 16):
            bigs[1][r, pl.ds(f * 16, 16)] = jnp.zeros((16,), jnp.float32)
        return carry

    lax.fori_loop(0, ZR, zstore, 0)
    for i in range(STRIPE // ZR):
        pltpu.sync_copy(bigs[1].at[pl.ds(0, ZR)],
                        acc_sh.at[pl.ds(row0 + i * ZR, ZR)])
    plsc.subcore_barrier()
    for i in range(PC):
        pltpu.async_copy(xs_sh.at[srcf_v.at[pl.ds(i * CH, CH)]],
                         bigs[0].at[pl.ds(i * CH, CH)], gsems[0])

    dn = lax.GatherDimensionNumbers(offset_dims=(), collapsed_slice_dims=(0,),
                                    start_index_map=(0,))

    def scale_phase(p, big):
        def cj_body(cl, carry):
            base = cl * CH
            for eb in range(CH // 16):
                w16 = w_v[p * PC + cl, pl.ds(eb * 16, 16)]
                for l in range(16):
                    wb = lax.gather(w16, jnp.full((16, 1), l, jnp.int32),
                                    dn, (1,),
                                    mode=lax.GatherScatterMode.PROMISE_IN_BOUNDS)
                    e = base + eb * 16 + l
                    for f in range(D // 16):
                        sl = pl.ds(f * 16, 16)
                        big[e, sl] = big[e, sl] * wb
            return carry

        lax.fori_loop(0, PC, cj_body, 0)

    def outer(k, carry):
        for q in range(2):
            p = k * 2 + q
            qn = 1 - q
            big, gs, ss = bigs[q], gsems[q], ssems[q]
            pltpu.make_async_copy(xs_sh.at[srcf_v.at[pl.ds(0, PE)]], big,
                                  gs).wait()

            @pl.when(p + 1 < NPH)
            def _issue_next():
                @pl.when(p >= 1)
                def _drain_prev():
                    for i in range(PC):
                        pltpu.make_async_copy(
                            bigs[qn].at[pl.ds(i * CH, CH)],
                            acc_sh.at[dst_v.at[0]],
                            ssems[qn]).wait()
                for i in range(PC):
                    pltpu.async_copy(
                        xs_sh.at[srcf_v.at[pl.ds((p + 1) * PE + i * CH, CH)]],
                        bigs[qn].at[pl.ds(i * CH, CH)], gsems[qn])

            scale_phase(p, big)
            for i in range(PC):
                pltpu.async_copy(big.at[pl.ds(i * CH, CH)],
                                 acc_sh.at[dst_v.at[p * PC + i]], ss,
                                 add=True)
        return carry

    lax.fori_loop(0, NPH // 2, outer, 0)
    for q in range(2):
        for i in range(PC):
            pltpu.make_async_copy(bigs[q].at[pl.ds(i * CH, CH)],
                                  acc_sh.at[dst_v.at[0]],
                                  ssems[q]).wait()
    plsc.subcore_barrier()
    pltpu.sync_copy(acc_sh.at[pl.ds(row0, STRIPE)],
                    out_hbm.at[cid, pl.ds(row0, STRIPE)])


def _sc_message_pass(xsp, srcf, dst3, w3, D, PC):
    PE = PC * CH
    def body(xsp_hbm, srcf_hbm, dst_hbm, w_hbm, out_hbm,
             sv, dv, wv, b0, b1, g0, g1, s0, s1, xs_sh, acc_sh):
        _mp_body(xsp_hbm, srcf_hbm, dst_hbm, w_hbm, out_hbm,
                 sv, dv, wv, (b0, b1), (g0, g1), (s0, s1), xs_sh, acc_sh,
                 D, PC)

    return pl.kernel(
        body,
        out_type=jax.ShapeDtypeStruct((NC, NP, D), jnp.float32),
        mesh=_MESH,
        compiler_params=pltpu.CompilerParams(use_tc_tiling_on_sc=False),
        scratch_types=[
            pltpu.VMEM((EPW,), jnp.int32),
            pltpu.VMEM((NCHUNK, CH), jnp.int32),
            pltpu.VMEM((NCHUNK, CH), jnp.float32),
        ] + [pltpu.VMEM((PE, D), jnp.float32)] * 2
          + [pltpu.SemaphoreType.DMA] * 4
          + [pltpu.VMEM_SHARED((N, D), jnp.float32),
             pltpu.VMEM_SHARED((NP, D), jnp.float32)],
    )(xsp, srcf, dst3, w3)


_RB = 1000


def _dinv_of(deg2_ref):
    d = deg2_ref[..., 0:1] + deg2_ref[..., 1:2] + 1.0
    return lax.rsqrt(d)


def _tc_mm_scale_body(deg2_ref, x_ref, w_ref, o_ref):
    xw = jnp.dot(x_ref[...], w_ref[...], preferred_element_type=jnp.float32)
    o_ref[...] = _dinv_of(deg2_ref) * xw


def _tc_mm_scale(deg2, x, w):
    n, k = x.shape
    m = w.shape[1]
    return pl.pallas_call(
        _tc_mm_scale_body,
        grid=(n // _RB,),
        in_specs=[pl.BlockSpec((_RB, 2), lambda i: (i, 0)),
                  pl.BlockSpec((_RB, k), lambda i: (i, 0)),
                  pl.BlockSpec((k, m), lambda i: (0, 0))],
        out_specs=pl.BlockSpec((_RB, m), lambda i: (i, 0)),
        out_shape=jax.ShapeDtypeStruct((n, m), jnp.float32),
    )(deg2, x, w)


def _tc_combine_body(deg2_ref, ysp_ref, xs_ref, b_ref, w_ref, o_ref):
    dinv = _dinv_of(deg2_ref)
    h = dinv * (ysp_ref[0] + ysp_ref[1] + xs_ref[...]) + b_ref[...]
    h = jnp.maximum(h, 0.0)
    hw = jnp.dot(h, w_ref[...], preferred_element_type=jnp.float32)
    o_ref[...] = dinv * hw


def _tc_combine_mm(deg2, ysp, xs, b, w):
    n, m = xs.shape
    mo = w.shape[1]
    return pl.pallas_call(
        _tc_combine_body,
        grid=(n // _RB,),
        in_specs=[pl.BlockSpec((_RB, 2), lambda i: (i, 0)),
                  pl.BlockSpec((NC, _RB, m), lambda i: (0, i, 0)),
                  pl.BlockSpec((_RB, m), lambda i: (i, 0)),
                  pl.BlockSpec((1, m), lambda i: (0, 0)),
                  pl.BlockSpec((m, mo), lambda i: (0, 0))],
        out_specs=pl.BlockSpec((_RB, mo), lambda i: (i, 0)),
        out_shape=jax.ShapeDtypeStruct((n, mo), jnp.float32),
    )(deg2, ysp, xs, b, w)


def _tc_final_body(deg2_ref, ysp_ref, xs_ref, b_ref, w_ref, b3_ref, o_ref):
    dinv = _dinv_of(deg2_ref)
    h = dinv * (ysp_ref[0] + ysp_ref[1] + xs_ref[...]) + b_ref[...]
    h = jnp.maximum(h, 0.0)
    o_ref[...] = (jnp.dot(h, w_ref[...], preferred_element_type=jnp.float32)
                  + b3_ref[...])


def _tc_final(deg2, ysp, xs, b, w, b3):
    n, m = xs.shape
    mo = w.shape[1]
    return pl.pallas_call(
        _tc_final_body,
        grid=(n // _RB,),
        in_specs=[pl.BlockSpec((_RB, 2), lambda i: (i, 0)),
                  pl.BlockSpec((NC, _RB, m), lambda i: (0, i, 0)),
                  pl.BlockSpec((_RB, m), lambda i: (i, 0)),
                  pl.BlockSpec((1, m), lambda i: (0, 0)),
                  pl.BlockSpec((m, mo), lambda i: (0, 0)),
                  pl.BlockSpec((1, mo), lambda i: (0, 0))],
        out_specs=pl.BlockSpec((_RB, mo), lambda i: (i, 0)),
        out_shape=jax.ShapeDtypeStruct((n, mo), jnp.float32),
    )(deg2, ysp, xs, b, w, b3)


def kernel(x, edge_index, edge_attr, W1, b1, W2, b2, W3, b3):
    src = edge_index[0].astype(jnp.int32)
    dst = edge_index[1].astype(jnp.int32)
    w = edge_attr.astype(jnp.float32)

    pad = EP - E
    src3 = jnp.concatenate([src, jnp.zeros((pad,), jnp.int32)]
                           ).reshape(NW, NCHUNK, CH)
    dst3 = jnp.concatenate([dst, jnp.zeros((pad,), jnp.int32)]
                           ).reshape(NW, NCHUNK, CH)
    w3 = jnp.concatenate([w, jnp.zeros((pad,), jnp.float32)]
                         ).reshape(NW, NCHUNK, CH)
    srcf = src3.reshape(NW, EPW)
    degp = _sc_degree(dst3, w3)
    deg2 = degp.T

    xs1 = _tc_mm_scale(deg2, x, W1)
    ys1 = _sc_message_pass(xs1, srcf, dst3, w3, 32, 4)
    xs2 = _tc_combine_mm(deg2, ys1, xs1, b1.reshape(1, 32), W2)
    ys2 = _sc_message_pass(xs2, srcf, dst3, w3, 64, 2)
    return _tc_final(deg2, ys2, xs2, b2.reshape(1, 64), W3, b3.reshape(1, 128))


# speedup vs baseline: 1.0012x; 1.0012x over previous
import jax
import jax.numpy as jnp
from jax import lax
from jax.experimental import pallas as pl
from jax.experimental.pallas import tpu as pltpu
from jax.experimental.pallas import tpu_sc as plsc

N = 10000
E = 160000
NP = 10240
NC = 2
NS = 16
NW = NC * NS
CH = 128
NCHUNK = 40
EPW = CH * NCHUNK
EP = EPW * NW
STRIPE = NP // NS

_MESH = plsc.VectorSubcoreMesh(core_axis_name="c", subcore_axis_name="s",
                               num_cores=NC, num_subcores=NS)


def _deg_body(dst_hbm, w_hbm, out_hbm, dst_v, w_v, zv, acc_sh, sem):
    cid = lax.axis_index("c")
    sid = lax.axis_index("s")
    wid = cid * NS + sid
    row0 = sid * STRIPE

    def zstore(r, carry):
        zv[pl.ds(r * 16, 16)] = jnp.zeros((16,), jnp.float32)
        return carry

    lax.fori_loop(0, STRIPE // 16, zstore, 0)
    pltpu.sync_copy(zv, acc_sh.at[pl.ds(row0, STRIPE)])
    pltpu.sync_copy(dst_hbm.at[wid], dst_v)
    pltpu.sync_copy(w_hbm.at[wid], w_v)
    plsc.subcore_barrier()
    def chunk(j, carry):
        pltpu.async_copy(w_v.at[j], acc_sh.at[dst_v.at[j]], sem, add=True)
        return carry

    lax.fori_loop(0, NCHUNK, chunk, 0)

    def drain(j, carry):
        pltpu.make_async_copy(w_v.at[0], acc_sh.at[dst_v.at[0]], sem).wait()
        return carry

    lax.fori_loop(0, NCHUNK, drain, 0)
    plsc.subcore_barrier()
    pltpu.sync_copy(acc_sh.at[pl.ds(row0, STRIPE)],
                    out_hbm.at[cid, pl.ds(row0, STRIPE)])


def _sc_degree(dst3, w3):
    return pl.kernel(
        _deg_body,
        out_type=jax.ShapeDtypeStruct((NC, NP), jnp.float32),
        mesh=_MESH,
        scratch_types=[
            pltpu.VMEM((NCHUNK, CH), jnp.int32),
            pltpu.VMEM((NCHUNK, CH), jnp.float32),
            pltpu.VMEM((STRIPE,), jnp.float32),
            pltpu.VMEM_SHARED((NP,), jnp.float32),
            pltpu.SemaphoreType.DMA,
        ],
    )(dst3, w3)


def _mp_body(xsp_hbm, srcf_hbm, dst_hbm, w_hbm, out_hbm,
             srcf_v, dst_v, w_v, bigs, gsems, ssems, xs_sh, acc_sh, D, PC):
    PE = PC * CH
    NPH = NCHUNK // PC
    cid = lax.axis_index("c")
    sid = lax.axis_index("s")
    wid = cid * NS + sid
    row0 = sid * STRIPE
    pltpu.sync_copy(srcf_hbm.at[wid], srcf_v)
    pltpu.sync_copy(dst_hbm.at[wid], dst_v)
    pltpu.sync_copy(w_hbm.at[wid], w_v)
    NSTR = N // NS
    pltpu.sync_copy(xsp_hbm.at[pl.ds(sid * NSTR, NSTR)],
                    xs_sh.at[pl.ds(sid * NSTR, NSTR)])
    ZR = 128

    def zstore(r, carry):
        for f in range(D // 16):
            bigs[1][r, pl.ds(f * 16, 16)] = jnp.zeros((16,), jnp.float32)
        return carry

    lax.fori_loop(0, ZR, zstore, 0)
    for i in range(STRIPE // ZR):
        pltpu.sync_copy(bigs[1].at[pl.ds(0, ZR)],
                        acc_sh.at[pl.ds(row0 + i * ZR, ZR)])
    plsc.subcore_barrier()
    for i in range(PC):
        pltpu.async_copy(xs_sh.at[srcf_v.at[pl.ds(i * CH, CH)]],
                         bigs[0].at[pl.ds(i * CH, CH)], gsems[0])

    dn = lax.GatherDimensionNumbers(offset_dims=(), collapsed_slice_dims=(0,),
                                    start_index_map=(0,))

    def scale_phase(p, big):
        def cj_body(cl, carry):
            base = cl * CH
            for eb in range(CH // 16):
                w16 = w_v[p * PC + cl, pl.ds(eb * 16, 16)]
                for l in range(16):
                    wb = lax.gather(w16, jnp.full((16, 1), l, jnp.int32),
                                    dn, (1,),
                                    mode=lax.GatherScatterMode.PROMISE_IN_BOUNDS)
                    e = base + eb * 16 + l
                    for f in range(D // 16):
                        sl = pl.ds(f * 16, 16)
                        big[e, sl] = big[e, sl] * wb
            return carry

        lax.fori_loop(0, PC, cj_body, 0)

    def outer(k, carry):
        for q in range(2):
            p = k * 2 + q
            qn = 1 - q
            big, gs, ss = bigs[q], gsems[q], ssems[q]
            pltpu.make_async_copy(xs_sh.at[srcf_v.at[pl.ds(0, PE)]], big,
                                  gs).wait()

            @pl.when(p + 1 < NPH)
            def _issue_next():
                @pl.when(p >= 1)
                def _drain_prev():
                    for i in range(PC):
                        pltpu.make_async_copy(
                            bigs[qn].at[pl.ds(i * CH, CH)],
                            acc_sh.at[dst_v.at[0]],
                            ssems[qn]).wait()
                for i in range(PC):
                    pltpu.async_copy(
                        xs_sh.at[srcf_v.at[pl.ds((p + 1) * PE + i * CH, CH)]],
                        bigs[qn].at[pl.ds(i * CH, CH)], gsems[qn])

            scale_phase(p, big)
            for i in range(PC):
                pltpu.async_copy(big.at[pl.ds(i * CH, CH)],
                                 acc_sh.at[dst_v.at[p * PC + i]], ss,
                                 add=True)
        return carry

    lax.fori_loop(0, NPH // 2, outer, 0)
    for q in range(2):
        for i in range(PC):
            pltpu.make_async_copy(bigs[q].at[pl.ds(i * CH, CH)],
                                  acc_sh.at[dst_v.at[0]],
                                  ssems[q]).wait()
    plsc.subcore_barrier()
    pltpu.sync_copy(acc_sh.at[pl.ds(row0, STRIPE)],
                    out_hbm.at[cid, pl.ds(row0, STRIPE)])


def _sc_message_pass(xsp, srcf, dst3, w3, D, PC):
    PE = PC * CH
    def body(xsp_hbm, srcf_hbm, dst_hbm, w_hbm, out_hbm,
             sv, dv, wv, b0, b1, g0, g1, s0, s1, xs_sh, acc_sh):
        _mp_body(xsp_hbm, srcf_hbm, dst_hbm, w_hbm, out_hbm,
                 sv, dv, wv, (b0, b1), (g0, g1), (s0, s1), xs_sh, acc_sh,
                 D, PC)

    return pl.kernel(
        body,
        out_type=jax.ShapeDtypeStruct((NC, NP, D), jnp.float32),
        mesh=_MESH,
        compiler_params=pltpu.CompilerParams(use_tc_tiling_on_sc=False),
        scratch_types=[
            pltpu.VMEM((EPW,), jnp.int32),
            pltpu.VMEM((NCHUNK, CH), jnp.int32),
            pltpu.VMEM((NCHUNK, CH), jnp.float32),
        ] + [pltpu.VMEM((PE, D), jnp.float32)] * 2
          + [pltpu.SemaphoreType.DMA] * 4
          + [pltpu.VMEM_SHARED((N, D), jnp.float32),
             pltpu.VMEM_SHARED((NP, D), jnp.float32)],
    )(xsp, srcf, dst3, w3)


_RB = 1000


def _dinv_of(deg2_ref):
    d = deg2_ref[..., 0:1] + deg2_ref[..., 1:2] + 1.0
    return lax.rsqrt(d)


def _tc_mm_scale_body(deg2_ref, x_ref, w_ref, o_ref):
    xw = jnp.dot(x_ref[...], w_ref[...], preferred_element_type=jnp.float32)
    o_ref[...] = _dinv_of(deg2_ref) * xw


def _tc_mm_scale(deg2, x, w):
    n, k = x.shape
    m = w.shape[1]
    return pl.pallas_call(
        _tc_mm_scale_body,
        grid=(n // _RB,),
        in_specs=[pl.BlockSpec((_RB, 2), lambda i: (i, 0)),
                  pl.BlockSpec((_RB, k), lambda i: (i, 0)),
                  pl.BlockSpec((k, m), lambda i: (0, 0))],
        out_specs=pl.BlockSpec((_RB, m), lambda i: (i, 0)),
        out_shape=jax.ShapeDtypeStruct((n, m), jnp.float32),
    )(deg2, x, w)


def _tc_combine_body(deg2_ref, ysp_ref, xs_ref, b_ref, w_ref, o_ref):
    dinv = _dinv_of(deg2_ref)
    h = dinv * (ysp_ref[0] + ysp_ref[1] + xs_ref[...]) + b_ref[...]
    h = jnp.maximum(h, 0.0)
    hw = jnp.dot(h, w_ref[...], preferred_element_type=jnp.float32)
    o_ref[...] = dinv * hw


def _tc_combine_mm(deg2, ysp, xs, b, w):
    n, m = xs.shape
    mo = w.shape[1]
    return pl.pallas_call(
        _tc_combine_body,
        grid=(n // _RB,),
        in_specs=[pl.BlockSpec((_RB, 2), lambda i: (i, 0)),
                  pl.BlockSpec((NC, _RB, m), lambda i: (0, i, 0)),
                  pl.BlockSpec((_RB, m), lambda i: (i, 0)),
                  pl.BlockSpec((1, m), lambda i: (0, 0)),
                  pl.BlockSpec((m, mo), lambda i: (0, 0))],
        out_specs=pl.BlockSpec((_RB, mo), lambda i: (i, 0)),
        out_shape=jax.ShapeDtypeStruct((n, mo), jnp.float32),
    )(deg2, ysp, xs, b, w)


def _tc_final_body(deg2_ref, ysp_ref, xs_ref, b_ref, w_ref, b3_ref, o_ref):
    dinv = _dinv_of(deg2_ref)
    h = dinv * (ysp_ref[0] + ysp_ref[1] + xs_ref[...]) + b_ref[...]
    h = jnp.maximum(h, 0.0)
    o_ref[...] = (jnp.dot(h, w_ref[...], preferred_element_type=jnp.float32)
                  + b3_ref[...])


def _tc_final(deg2, ysp, xs, b, w, b3):
    n, m = xs.shape
    mo = w.shape[1]
    return pl.pallas_call(
        _tc_final_body,
        grid=(n // _RB,),
        in_specs=[pl.BlockSpec((_RB, 2), lambda i: (i, 0)),
                  pl.BlockSpec((NC, _RB, m), lambda i: (0, i, 0)),
                  pl.BlockSpec((_RB, m), lambda i: (i, 0)),
                  pl.BlockSpec((1, m), lambda i: (0, 0)),
                  pl.BlockSpec((m, mo), lambda i: (0, 0)),
                  pl.BlockSpec((1, mo), lambda i: (0, 0))],
        out_specs=pl.BlockSpec((_RB, mo), lambda i: (i, 0)),
        out_shape=jax.ShapeDtypeStruct((n, mo), jnp.float32),
    )(deg2, ysp, xs, b, w, b3)


def kernel(x, edge_index, edge_attr, W1, b1, W2, b2, W3, b3):
    src = edge_index[0].astype(jnp.int32)
    dst = edge_index[1].astype(jnp.int32)
    w = edge_attr.astype(jnp.float32)

    pad = EP - E
    src3 = jnp.concatenate([src, jnp.zeros((pad,), jnp.int32)]
                           ).reshape(NW, NCHUNK, CH)
    dst3 = jnp.concatenate([dst, jnp.zeros((pad,), jnp.int32)]
                           ).reshape(NW, NCHUNK, CH)
    w3 = jnp.concatenate([w, jnp.zeros((pad,), jnp.float32)]
                         ).reshape(NW, NCHUNK, CH)
    srcf = src3.reshape(NW, EPW)
    degp = _sc_degree(dst3, w3)
    deg2 = degp.T

    xs1 = _tc_mm_scale(deg2, x, W1)
    ys1 = _sc_message_pass(xs1, srcf, dst3, w3, 32, 4)
    xs2 = _tc_combine_mm(deg2, ys1, xs1, b1.reshape(1, 32), W2)
    ys2 = _sc_message_pass(xs2, srcf, dst3, w3, 64, 2)
    return _tc_final(deg2, ys2, xs2, b2.reshape(1, 64), W3, b3.reshape(1, 128))
